# speedup vs baseline: 1.6307x; 1.6307x over previous
import functools

import jax
import jax.numpy as jnp
from jax import lax
from jax.experimental import pallas as pl
from jax.experimental.pallas import tpu as pltpu
from jax.experimental.pallas import tpu_sc as plsc

VOCAB = 100000
EMBED_DIM = 1024
BATCH = 4
SEQ_LEN = 4096

_INFO = plsc.get_sparse_core_info()
NC, NS = _INFO.num_cores, _INFO.num_subcores
NW = NC * NS
TOTAL = BATCH * SEQ_LEN
B_PER_W = TOTAL // NW
CHUNK = 32
N_CHUNKS = B_PER_W // CHUNK


def _gather_body(table_hbm, idx_hbm, out_hbm, idx_v, rows_v, gsem, osem):
    wid = lax.axis_index("s") * NC + lax.axis_index("c")
    base = wid * B_PER_W

    pltpu.sync_copy(idx_hbm.at[wid], idx_v)

    pltpu.async_copy(table_hbm.at[idx_v.at[0]], rows_v.at[0], gsem.at[0])

    for j in range(N_CHUNKS):
        cb = j % 2
        nb = (j + 1) % 2
        if j + 1 < N_CHUNKS:
            if j >= 1:
                pltpu.make_async_copy(
                    rows_v.at[nb],
                    out_hbm.at[pl.ds(base + (j - 1) * CHUNK, CHUNK)],
                    osem.at[nb]).wait()
            pltpu.async_copy(
                table_hbm.at[idx_v.at[j + 1]], rows_v.at[nb], gsem.at[nb])
        pltpu.make_async_copy(
            table_hbm.at[idx_v.at[j]], rows_v.at[cb], gsem.at[cb]).wait()
        pltpu.async_copy(
            rows_v.at[cb], out_hbm.at[pl.ds(base + j * CHUNK, CHUNK)],
            osem.at[cb])

    pltpu.make_async_copy(
        rows_v.at[(N_CHUNKS - 2) % 2],
        out_hbm.at[pl.ds(base + (N_CHUNKS - 2) * CHUNK, CHUNK)],
        osem.at[(N_CHUNKS - 2) % 2]).wait()
    pltpu.make_async_copy(
        rows_v.at[(N_CHUNKS - 1) % 2],
        out_hbm.at[pl.ds(base + (N_CHUNKS - 1) * CHUNK, CHUNK)],
        osem.at[(N_CHUNKS - 1) % 2]).wait()


@jax.jit
def kernel(input_ids, embedding_table):
    idx = input_ids.reshape(NW, N_CHUNKS, CHUNK).astype(jnp.int32)
    mesh = plsc.VectorSubcoreMesh(core_axis_name="c", subcore_axis_name="s")
    out = pl.kernel(
        _gather_body,
        out_type=jax.ShapeDtypeStruct((TOTAL, EMBED_DIM), jnp.float32),
        mesh=mesh,
        scratch_types=[
            pltpu.VMEM((N_CHUNKS, CHUNK), jnp.int32),
            pltpu.VMEM((2, CHUNK, EMBED_DIM), jnp.float32),
            pltpu.SemaphoreType.DMA((2,)),
            pltpu.SemaphoreType.DMA((2,)),
        ],
    )(embedding_table, idx)
    return out.reshape(BATCH, SEQ_LEN, EMBED_DIM)


# speedup vs baseline: 1.6519x; 1.0130x over previous
import functools

import jax
import jax.numpy as jnp
from jax import lax
from jax.experimental import pallas as pl
from jax.experimental.pallas import tpu as pltpu
from jax.experimental.pallas import tpu_sc as plsc

VOCAB = 100000
EMBED_DIM = 1024
BATCH = 4
SEQ_LEN = 4096

_INFO = plsc.get_sparse_core_info()
NC, NS = _INFO.num_cores, _INFO.num_subcores
NW = NC * NS
TOTAL = BATCH * SEQ_LEN
B_PER_W = TOTAL // NW
CHUNK = 32
N_CHUNKS = B_PER_W // CHUNK
NBUF = 3


def _gather_body(table_hbm, idx_hbm, out_hbm, idx_v, rows_v, gsem, osem):
    wid = lax.axis_index("s") * NC + lax.axis_index("c")
    base = wid * B_PER_W

    pltpu.sync_copy(idx_hbm.at[wid], idx_v)

    for b in range(NBUF - 1):
        pltpu.async_copy(table_hbm.at[idx_v.at[b]], rows_v.at[b], gsem.at[b])

    for j in range(N_CHUNKS):
        b = j % NBUF
        nxt = j + NBUF - 1
        if nxt < N_CHUNKS:
            bn = nxt % NBUF
            if nxt >= NBUF:
                pltpu.make_async_copy(
                    rows_v.at[bn],
                    out_hbm.at[pl.ds(base + (nxt - NBUF) * CHUNK, CHUNK)],
                    osem.at[bn]).wait()
            pltpu.async_copy(
                table_hbm.at[idx_v.at[nxt]], rows_v.at[bn], gsem.at[bn])
        pltpu.make_async_copy(
            table_hbm.at[idx_v.at[j]], rows_v.at[b], gsem.at[b]).wait()
        pltpu.async_copy(
            rows_v.at[b], out_hbm.at[pl.ds(base + j * CHUNK, CHUNK)],
            osem.at[b])

    for c in range(N_CHUNKS - NBUF, N_CHUNKS):
        pltpu.make_async_copy(
            rows_v.at[c % NBUF],
            out_hbm.at[pl.ds(base + c * CHUNK, CHUNK)],
            osem.at[c % NBUF]).wait()


@jax.jit
def kernel(input_ids, embedding_table):
    idx = input_ids.reshape(NW, N_CHUNKS, CHUNK).astype(jnp.int32)
    mesh = plsc.VectorSubcoreMesh(core_axis_name="c", subcore_axis_name="s")
    out = pl.kernel(
        _gather_body,
        out_type=jax.ShapeDtypeStruct((TOTAL, EMBED_DIM), jnp.float32),
        mesh=mesh,
        scratch_types=[
            pltpu.VMEM((N_CHUNKS, CHUNK), jnp.int32),
            pltpu.VMEM((NBUF, CHUNK, EMBED_DIM), jnp.float32),
            pltpu.SemaphoreType.DMA((NBUF,)),
            pltpu.SemaphoreType.DMA((NBUF,)),
        ],
    )(embedding_table, idx)
    return out.reshape(BATCH, SEQ_LEN, EMBED_DIM)


# speedup vs baseline: 1.6581x; 1.0038x over previous
import functools

import jax
import jax.numpy as jnp
from jax import lax
from jax.experimental import pallas as pl
from jax.experimental.pallas import tpu as pltpu
from jax.experimental.pallas import tpu_sc as plsc

VOCAB = 100000
EMBED_DIM = 1024
BATCH = 4
SEQ_LEN = 4096

_INFO = plsc.get_sparse_core_info()
NC, NS = _INFO.num_cores, _INFO.num_subcores
NW = NC * NS
TOTAL = BATCH * SEQ_LEN
B_PER_W = TOTAL // NW
CHUNK = 16
N_CHUNKS = B_PER_W // CHUNK
NBUF = 6


def _gather_body(table_hbm, idx_hbm, out_hbm, idx_v, rows_v, gsem, osem):
    wid = lax.axis_index("s") * NC + lax.axis_index("c")
    base = wid * B_PER_W

    pltpu.sync_copy(idx_hbm.at[wid], idx_v)

    for b in range(NBUF - 1):
        pltpu.async_copy(table_hbm.at[idx_v.at[b]], rows_v.at[b], gsem.at[b])

    for j in range(N_CHUNKS):
        b = j % NBUF
        nxt = j + NBUF - 1
        if nxt < N_CHUNKS:
            bn = nxt % NBUF
            if nxt >= NBUF:
                pltpu.make_async_copy(
                    rows_v.at[bn],
                    out_hbm.at[pl.ds(base + (nxt - NBUF) * CHUNK, CHUNK)],
                    osem.at[bn]).wait()
            pltpu.async_copy(
                table_hbm.at[idx_v.at[nxt]], rows_v.at[bn], gsem.at[bn])
        pltpu.make_async_copy(
            table_hbm.at[idx_v.at[j]], rows_v.at[b], gsem.at[b]).wait()
        pltpu.async_copy(
            rows_v.at[b], out_hbm.at[pl.ds(base + j * CHUNK, CHUNK)],
            osem.at[b])

    for c in range(N_CHUNKS - NBUF, N_CHUNKS):
        pltpu.make_async_copy(
            rows_v.at[c % NBUF],
            out_hbm.at[pl.ds(base + c * CHUNK, CHUNK)],
            osem.at[c % NBUF]).wait()


@jax.jit
def kernel(input_ids, embedding_table):
    idx = input_ids.reshape(NW, N_CHUNKS, CHUNK).astype(jnp.int32)
    mesh = plsc.VectorSubcoreMesh(core_axis_name="c", subcore_axis_name="s")
    out = pl.kernel(
        _gather_body,
        out_type=jax.ShapeDtypeStruct((TOTAL, EMBED_DIM), jnp.float32),
        mesh=mesh,
        scratch_types=[
            pltpu.VMEM((N_CHUNKS, CHUNK), jnp.int32),
            pltpu.VMEM((NBUF, CHUNK, EMBED_DIM), jnp.float32),
            pltpu.SemaphoreType.DMA((NBUF,)),
            pltpu.SemaphoreType.DMA((NBUF,)),
        ],
    )(embedding_table, idx)
    return out.reshape(BATCH, SEQ_LEN, EMBED_DIM)
